# baseline (speedup 1.0000x reference)
import jax
import jax.numpy as jnp
from jax.experimental import pallas as pl
from jax.experimental.pallas import tpu as pltpu

NHID = 128
QUEUE_SIZE = 16384
NQUEUE = 16
W = 2048


def _body(qid_ref, elem_ref, qptr_smem, k_ref, q_in, ids_in, qptr_v,
          q_out, ids_out, ptr_out):
    q = pl.program_id(0)
    c = pl.program_id(1)
    qid = qid_ref[0]
    ptr = qptr_smem[qid]

    q_out[...] = q_in[...]
    ids_out[...] = ids_in[...]

    local = ptr - c * W
    hit = (q == qid) & (local >= 0) & (local < W)

    @pl.when(hit)
    def _patch():
        lane3 = jax.lax.broadcasted_iota(jnp.int32, (1, NHID, W), 2)
        kb = k_ref[...].reshape(1, NHID, 1)
        q_out[...] = jnp.where(lane3 == local, kb, q_in[...])
        lane_ids = jax.lax.broadcasted_iota(jnp.int32, (1, 1, W), 2)
        ids_out[...] = jnp.where(lane_ids == local, elem_ref[0], ids_in[...])

    lane = jax.lax.broadcasted_iota(jnp.int32, (1, NQUEUE), 1)
    ptr_out[...] = jnp.where(lane == qid, (ptr + 1) % QUEUE_SIZE, qptr_v[...])


def kernel(k, queue, ids, queue_ptr, elem_id, q_id):
    qid = jnp.asarray(q_id, jnp.int32).reshape(1)
    elem = jnp.asarray(elem_id, jnp.int32).reshape(1)
    k2 = k.reshape(NHID, 1)
    ids3 = ids.reshape(NQUEUE, 1, QUEUE_SIZE)
    qptr2 = queue_ptr.reshape(1, NQUEUE)

    grid = (NQUEUE, QUEUE_SIZE // W)
    out_q, out_ids, out_ptr = pl.pallas_call(
        _body,
        grid=grid,
        in_specs=[
            pl.BlockSpec(memory_space=pltpu.SMEM),
            pl.BlockSpec(memory_space=pltpu.SMEM),
            pl.BlockSpec(memory_space=pltpu.SMEM),
            pl.BlockSpec((NHID, 1), lambda q, c: (0, 0)),
            pl.BlockSpec((1, NHID, W), lambda q, c: (q, 0, c)),
            pl.BlockSpec((1, 1, W), lambda q, c: (q, 0, c)),
            pl.BlockSpec((1, NQUEUE), lambda q, c: (0, 0)),
        ],
        out_specs=[
            pl.BlockSpec((1, NHID, W), lambda q, c: (q, 0, c)),
            pl.BlockSpec((1, 1, W), lambda q, c: (q, 0, c)),
            pl.BlockSpec((1, NQUEUE), lambda q, c: (0, 0)),
        ],
        out_shape=[
            jax.ShapeDtypeStruct((NQUEUE, NHID, QUEUE_SIZE), jnp.float32),
            jax.ShapeDtypeStruct((NQUEUE, 1, QUEUE_SIZE), jnp.int32),
            jax.ShapeDtypeStruct((1, NQUEUE), jnp.int32),
        ],
        compiler_params=pltpu.CompilerParams(
            dimension_semantics=("arbitrary", "arbitrary"),
        ),
    )(qid, elem, queue_ptr, k2, queue, ids3, qptr2)
    return out_q, out_ids.reshape(NQUEUE, QUEUE_SIZE), out_ptr.reshape(NQUEUE)


# speedup vs baseline: 1.4732x; 1.4732x over previous
import jax
import jax.numpy as jnp
from jax.experimental import pallas as pl
from jax.experimental.pallas import tpu as pltpu

NHID = 128
QUEUE_SIZE = 16384
NQUEUE = 16
W = 8192


def _body(qid_ref, elem_ref, qptr_smem, k_ref, q_in, ids_in, qptr_v,
          q_out, ids_out, ptr_out):
    q = pl.program_id(0)
    c = pl.program_id(1)
    qid = qid_ref[0]
    ptr = qptr_smem[qid]

    q_out[...] = q_in[...]

    local = ptr - c * W
    hit = (q == qid) & (local >= 0) & (local < W)

    @pl.when(hit)
    def _patch():
        lane3 = jax.lax.broadcasted_iota(jnp.int32, (1, NHID, W), 2)
        kb = k_ref[...].reshape(1, NHID, 1)
        q_out[...] = jnp.where(lane3 == local, kb, q_in[...])

    @pl.when((q == 0) & (c == 0))
    def _small():
        lane_ids = jax.lax.broadcasted_iota(jnp.int32, (NQUEUE, 1, QUEUE_SIZE), 2)
        row_ids = jax.lax.broadcasted_iota(jnp.int32, (NQUEUE, 1, QUEUE_SIZE), 0)
        ids_out[...] = jnp.where((lane_ids == ptr) & (row_ids == qid),
                                 elem_ref[0], ids_in[...])
        lane = jax.lax.broadcasted_iota(jnp.int32, (1, NQUEUE), 1)
        ptr_out[...] = jnp.where(lane == qid, (ptr + 1) % QUEUE_SIZE, qptr_v[...])


def kernel(k, queue, ids, queue_ptr, elem_id, q_id):
    qid = jnp.asarray(q_id, jnp.int32).reshape(1)
    elem = jnp.asarray(elem_id, jnp.int32).reshape(1)
    k2 = k.reshape(NHID, 1)
    ids3 = ids.reshape(NQUEUE, 1, QUEUE_SIZE)
    qptr2 = queue_ptr.reshape(1, NQUEUE)

    grid = (NQUEUE, QUEUE_SIZE // W)
    out_q, out_ids, out_ptr = pl.pallas_call(
        _body,
        grid=grid,
        in_specs=[
            pl.BlockSpec(memory_space=pltpu.SMEM),
            pl.BlockSpec(memory_space=pltpu.SMEM),
            pl.BlockSpec(memory_space=pltpu.SMEM),
            pl.BlockSpec((NHID, 1), lambda q, c: (0, 0)),
            pl.BlockSpec((1, NHID, W), lambda q, c: (q, 0, c)),
            pl.BlockSpec((NQUEUE, 1, QUEUE_SIZE), lambda q, c: (0, 0, 0)),
            pl.BlockSpec((1, NQUEUE), lambda q, c: (0, 0)),
        ],
        out_specs=[
            pl.BlockSpec((1, NHID, W), lambda q, c: (q, 0, c)),
            pl.BlockSpec((NQUEUE, 1, QUEUE_SIZE), lambda q, c: (0, 0, 0)),
            pl.BlockSpec((1, NQUEUE), lambda q, c: (0, 0)),
        ],
        out_shape=[
            jax.ShapeDtypeStruct((NQUEUE, NHID, QUEUE_SIZE), jnp.float32),
            jax.ShapeDtypeStruct((NQUEUE, 1, QUEUE_SIZE), jnp.int32),
            jax.ShapeDtypeStruct((1, NQUEUE), jnp.int32),
        ],
        compiler_params=pltpu.CompilerParams(
            dimension_semantics=("parallel", "parallel"),
        ),
    )(qid, elem, queue_ptr, k2, queue, ids3, qptr2)
    return out_q, out_ids.reshape(NQUEUE, QUEUE_SIZE), out_ptr.reshape(NQUEUE)


# speedup vs baseline: 1.5126x; 1.0267x over previous
import jax
import jax.numpy as jnp
from jax.experimental import pallas as pl
from jax.experimental.pallas import tpu as pltpu

NHID = 128
QUEUE_SIZE = 16384
NQUEUE = 16
W = 16384


def _body(qid_ref, elem_ref, qptr_smem, k_ref, q_in, ids_in, qptr_v,
          q_out, ids_out, ptr_out):
    q = pl.program_id(0)
    c = pl.program_id(1)
    qid = qid_ref[0]
    ptr = qptr_smem[qid]

    q_out[...] = q_in[...]

    local = ptr - c * W
    hit = (q == qid) & (local >= 0) & (local < W)

    @pl.when(hit)
    def _patch():
        lane3 = jax.lax.broadcasted_iota(jnp.int32, (1, NHID, W), 2)
        kb = k_ref[...].reshape(1, NHID, 1)
        q_out[...] = jnp.where(lane3 == local, kb, q_in[...])

    @pl.when((q == 0) & (c == 0))
    def _small():
        lane_ids = jax.lax.broadcasted_iota(jnp.int32, (NQUEUE, 1, QUEUE_SIZE), 2)
        row_ids = jax.lax.broadcasted_iota(jnp.int32, (NQUEUE, 1, QUEUE_SIZE), 0)
        ids_out[...] = jnp.where((lane_ids == ptr) & (row_ids == qid),
                                 elem_ref[0], ids_in[...])
        lane = jax.lax.broadcasted_iota(jnp.int32, (1, NQUEUE), 1)
        ptr_out[...] = jnp.where(lane == qid, (ptr + 1) % QUEUE_SIZE, qptr_v[...])


def kernel(k, queue, ids, queue_ptr, elem_id, q_id):
    qid = jnp.asarray(q_id, jnp.int32).reshape(1)
    elem = jnp.asarray(elem_id, jnp.int32).reshape(1)
    k2 = k.reshape(NHID, 1)
    ids3 = ids.reshape(NQUEUE, 1, QUEUE_SIZE)
    qptr2 = queue_ptr.reshape(1, NQUEUE)

    grid = (NQUEUE, QUEUE_SIZE // W)
    out_q, out_ids, out_ptr = pl.pallas_call(
        _body,
        grid=grid,
        in_specs=[
            pl.BlockSpec(memory_space=pltpu.SMEM),
            pl.BlockSpec(memory_space=pltpu.SMEM),
            pl.BlockSpec(memory_space=pltpu.SMEM),
            pl.BlockSpec((NHID, 1), lambda q, c: (0, 0)),
            pl.BlockSpec((1, NHID, W), lambda q, c: (q, 0, c)),
            pl.BlockSpec((NQUEUE, 1, QUEUE_SIZE), lambda q, c: (0, 0, 0)),
            pl.BlockSpec((1, NQUEUE), lambda q, c: (0, 0)),
        ],
        out_specs=[
            pl.BlockSpec((1, NHID, W), lambda q, c: (q, 0, c)),
            pl.BlockSpec((NQUEUE, 1, QUEUE_SIZE), lambda q, c: (0, 0, 0)),
            pl.BlockSpec((1, NQUEUE), lambda q, c: (0, 0)),
        ],
        out_shape=[
            jax.ShapeDtypeStruct((NQUEUE, NHID, QUEUE_SIZE), jnp.float32),
            jax.ShapeDtypeStruct((NQUEUE, 1, QUEUE_SIZE), jnp.int32),
            jax.ShapeDtypeStruct((1, NQUEUE), jnp.int32),
        ],
        compiler_params=pltpu.CompilerParams(
            dimension_semantics=("parallel", "parallel"),
        ),
    )(qid, elem, queue_ptr, k2, queue, ids3, qptr2)
    return out_q, out_ids.reshape(NQUEUE, QUEUE_SIZE), out_ptr.reshape(NQUEUE)
